# Initial kernel scaffold; baseline (speedup 1.0000x reference)
import jax
import jax.numpy as jnp
from jax.experimental import pallas as pl


def kernel(w_idxs, c_idxs, word_table, char_table, word_proj, char_conv_w, char_conv_b):
    raise NotImplementedError("write your pallas kernel here")


# speedup vs baseline: 1.1368x; 1.1368x over previous
import functools

import jax
import jax.numpy as jnp
from jax import lax
from jax.experimental import pallas as pl
from jax.experimental.pallas import tpu as pltpu
from jax.experimental.pallas import tpu_sc as plsc

VOCAB = 100000
WORD_DIM = 300
CHAR_VOCAB = 262
CHAR_DIM = 64
HIDDEN = 128
H2 = HIDDEN // 2
B = 1024
L = 200
W = 16
K = 5
T = W - K + 1

N = B * L

NC = 2
NS = 16
NW = NC * NS
LANES = 16

TOK_PER_W = N // NW
NB = 64
NCHUNK = TOK_PER_W // NB
NG = NB // LANES

ROWS_PCT = K * CHAR_VOCAB


def _pw_body(wt_ref, wp_ref, o_ref):
    o_ref[...] = jnp.dot(wt_ref[...], wp_ref[...],
                         preferred_element_type=jnp.float32)


def _project_word(word_table, word_proj):
    rows = 1000
    return pl.pallas_call(
        _pw_body,
        grid=(VOCAB // rows,),
        in_specs=[
            pl.BlockSpec((rows, WORD_DIM), lambda i: (i, 0)),
            pl.BlockSpec((WORD_DIM, H2), lambda i: (0, 0)),
        ],
        out_specs=pl.BlockSpec((rows, H2), lambda i: (i, 0)),
        out_shape=jax.ShapeDtypeStruct((VOCAB, H2), jnp.float32),
    )(word_table, word_proj)


def _pct_body(ct_ref, w_ref, b_ref, o_ref):
    k = pl.program_id(0)
    acc = jnp.dot(ct_ref[...], w_ref[0], preferred_element_type=jnp.float32)
    scale = jnp.where(k == 0, 1.0, 0.0)
    o_ref[0] = acc + scale * b_ref[...]


def _char_tables(char_table, char_conv_w, char_conv_b):
    out = pl.pallas_call(
        _pct_body,
        grid=(K,),
        in_specs=[
            pl.BlockSpec((CHAR_VOCAB, CHAR_DIM), lambda k: (0, 0)),
            pl.BlockSpec((1, CHAR_DIM, H2), lambda k: (k, 0, 0)),
            pl.BlockSpec((1, H2), lambda k: (0, 0)),
        ],
        out_specs=pl.BlockSpec((1, CHAR_VOCAB, H2), lambda k: (k, 0, 0)),
        out_shape=jax.ShapeDtypeStruct((K, CHAR_VOCAB, H2), jnp.float32),
    )(char_table, char_conv_w, char_conv_b.reshape(1, H2))
    return out.reshape(ROWS_PCT, H2)


@functools.cache
def _build_sc_main():
    mesh = plsc.VectorSubcoreMesh(core_axis_name="c", subcore_axis_name="s",
                                  num_cores=NC, num_subcores=NS)
    return pl.kernel(
        _sc_body,
        out_type=(jax.ShapeDtypeStruct((N, H2), jnp.float32),
                  jax.ShapeDtypeStruct((N * H2,), jnp.float32)),
        mesh=mesh,
        scratch_types=[
            pltpu.VMEM((ROWS_PCT * H2,), jnp.float32),
            pltpu.VMEM((NB,), jnp.int32),
            pltpu.VMEM((NB * W,), jnp.int32),
            pltpu.VMEM((LANES * LANES,), jnp.int32),
            pltpu.VMEM((NB, H2), jnp.float32),
            pltpu.VMEM((NB * H2,), jnp.float32),
            pltpu.SemaphoreType.DMA,
        ],
        compiler_params=pltpu.CompilerParams(use_tc_tiling_on_sc=False,
                                             needs_layout_passes=False),
    )


def _sc_body(wflat_hbm, cflat_hbm, pw_hbm, pct_hbm, outw_hbm, outc_hbm,
             pct_v, widx_v, cidx_v, ct_v, wrows_v, cbuf_v, sem):
    wid = lax.axis_index("s") * NC + lax.axis_index("c")
    pltpu.sync_copy(pct_hbm, pct_v)
    iota = lax.iota(jnp.int32, LANES)

    def chunk_body(ci, carry):
        base = wid * TOK_PER_W + ci * NB
        pltpu.sync_copy(wflat_hbm.at[pl.ds(base, NB)], widx_v)
        pltpu.sync_copy(cflat_hbm.at[pl.ds(base * W, NB * W)], cidx_v)
        pltpu.async_copy(pw_hbm.at[widx_v], wrows_v, sem).wait()

        def group_body(g, carry2):
            for i in range(LANES):
                chars = cidx_v[pl.ds((g * LANES + i) * W, W)]
                plsc.store_scatter(ct_v, [iota * LANES + i], chars)
            cvec = [ct_v[pl.ds(j * LANES, LANES)] * H2 for j in range(W)]
            row = (g * LANES + iota) * H2

            def h_body(h, carry3):
                m = None
                for t in range(T):
                    s = plsc.load_gather(pct_v, [cvec[t] + h])
                    for k in range(1, K):
                        s = s + plsc.load_gather(
                            pct_v, [cvec[t + k] + (k * CHAR_VOCAB * H2 + h)])
                    s = jnp.maximum(s, 0.0)
                    m = s if m is None else jnp.maximum(m, s)
                plsc.store_scatter(cbuf_v, [row + h], m)
                return carry3

            lax.fori_loop(0, H2, h_body, 0)
            return carry2

        lax.fori_loop(0, NG, group_body, 0)

        pltpu.sync_copy(wrows_v, outw_hbm.at[pl.ds(base, NB)])
        pltpu.sync_copy(cbuf_v, outc_hbm.at[pl.ds(base * H2, NB * H2)])
        return carry

    lax.fori_loop(0, NCHUNK, chunk_body, 0)


def kernel(w_idxs, c_idxs, word_table, char_table, word_proj,
           char_conv_w, char_conv_b):
    pw = _project_word(word_table, word_proj)
    pct = _char_tables(char_table, char_conv_w, char_conv_b)
    out_w, out_c = _build_sc_main()(w_idxs.reshape(-1), c_idxs.reshape(-1),
                                    pw, pct.reshape(-1))
    return jnp.concatenate([out_w.reshape(B, L, H2),
                            out_c.reshape(B, L, H2)], axis=-1)


# speedup vs baseline: 7.3512x; 6.4663x over previous
import functools

import jax
import jax.numpy as jnp
from jax import lax
from jax.experimental import pallas as pl
from jax.experimental.pallas import tpu as pltpu
from jax.experimental.pallas import tpu_sc as plsc

VOCAB = 100000
WORD_DIM = 300
CHAR_VOCAB = 262
CHAR_DIM = 64
HIDDEN = 128
H2 = HIDDEN // 2
B = 1024
L = 200
W = 16
K = 5
T = W - K + 1

N = B * L

NC = 2
NS = 16
NW = NC * NS
LANES = 16

TOK_PER_W = N // NW
NB = 64
NCHUNK = TOK_PER_W // NB
NG = NB // LANES

ROWS_PCT = K * CHAR_VOCAB
PCT_STRIDE = H2 + 1
CT_STRIDE = LANES + 1


def _pw_body(wt_ref, wp_ref, o_ref):
    o_ref[...] = jnp.dot(wt_ref[...], wp_ref[...],
                         preferred_element_type=jnp.float32)


def _project_word(word_table, word_proj):
    rows = 1000
    return pl.pallas_call(
        _pw_body,
        grid=(VOCAB // rows,),
        in_specs=[
            pl.BlockSpec((rows, WORD_DIM), lambda i: (i, 0)),
            pl.BlockSpec((WORD_DIM, H2), lambda i: (0, 0)),
        ],
        out_specs=pl.BlockSpec((rows, H2), lambda i: (i, 0)),
        out_shape=jax.ShapeDtypeStruct((VOCAB, H2), jnp.float32),
    )(word_table, word_proj)


def _pct_body(ct_ref, w_ref, b_ref, o_ref):
    k = pl.program_id(0)
    acc = jnp.dot(ct_ref[...], w_ref[0], preferred_element_type=jnp.float32)
    scale = jnp.where(k == 0, 1.0, 0.0)
    o_ref[0] = acc + scale * b_ref[...]


def _char_tables(char_table, char_conv_w, char_conv_b):
    out = pl.pallas_call(
        _pct_body,
        grid=(K,),
        in_specs=[
            pl.BlockSpec((CHAR_VOCAB, CHAR_DIM), lambda k: (0, 0)),
            pl.BlockSpec((1, CHAR_DIM, H2), lambda k: (k, 0, 0)),
            pl.BlockSpec((1, H2), lambda k: (0, 0)),
        ],
        out_specs=pl.BlockSpec((1, CHAR_VOCAB, H2), lambda k: (k, 0, 0)),
        out_shape=jax.ShapeDtypeStruct((K, CHAR_VOCAB, H2), jnp.float32),
    )(char_table, char_conv_w, char_conv_b.reshape(1, H2))
    return out.reshape(ROWS_PCT, H2)


@functools.cache
def _build_sc_main():
    mesh = plsc.VectorSubcoreMesh(core_axis_name="c", subcore_axis_name="s",
                                  num_cores=NC, num_subcores=NS)
    return pl.kernel(
        _sc_body,
        out_type=(jax.ShapeDtypeStruct((N, H2), jnp.float32),
                  jax.ShapeDtypeStruct((N * H2,), jnp.float32)),
        mesh=mesh,
        scratch_types=[
            pltpu.VMEM((ROWS_PCT * PCT_STRIDE,), jnp.float32),
            pltpu.VMEM((NB,), jnp.int32),
            pltpu.VMEM((NB * W,), jnp.int32),
            pltpu.VMEM((LANES * CT_STRIDE,), jnp.int32),
            pltpu.VMEM((NB, H2), jnp.float32),
            pltpu.VMEM((NB * H2,), jnp.float32),
            pltpu.SemaphoreType.DMA,
        ],
        compiler_params=pltpu.CompilerParams(use_tc_tiling_on_sc=False,
                                             needs_layout_passes=False),
    )


def _sc_body(wflat_hbm, cflat_hbm, pw_hbm, pct_hbm, outw_hbm, outc_hbm,
             pct_v, widx_v, cidx_v, ct_v, wrows_v, cbuf_v, sem):
    wid = lax.axis_index("s") * NC + lax.axis_index("c")
    pltpu.sync_copy(pct_hbm, pct_v)
    iota = lax.iota(jnp.int32, LANES)

    def chunk_body(ci, carry):
        base = wid * TOK_PER_W + ci * NB
        pltpu.sync_copy(wflat_hbm.at[pl.ds(base, NB)], widx_v)
        pltpu.sync_copy(cflat_hbm.at[pl.ds(base * W, NB * W)], cidx_v)
        pltpu.async_copy(pw_hbm.at[widx_v], wrows_v, sem).wait()

        def group_body(g, carry2):
            for i in range(LANES):
                chars = cidx_v[pl.ds((g * LANES + i) * W, W)]
                plsc.store_scatter(ct_v, [iota * CT_STRIDE + i], chars)
            cvec = [ct_v[pl.ds(j * CT_STRIDE, LANES)] * PCT_STRIDE
                    for j in range(W)]
            row = (g * LANES + iota) * H2

            def h_body(h, carry3):
                m = None
                for t in range(T):
                    s = plsc.load_gather(pct_v, [cvec[t] + h])
                    for k in range(1, K):
                        s = s + plsc.load_gather(
                            pct_v,
                            [cvec[t + k] + (k * CHAR_VOCAB * PCT_STRIDE + h)])
                    s = jnp.maximum(s, 0.0)
                    m = s if m is None else jnp.maximum(m, s)
                plsc.store_scatter(cbuf_v, [row + h], m)
                return carry3

            lax.fori_loop(0, H2, h_body, 0)
            return carry2

        lax.fori_loop(0, NG, group_body, 0)

        pltpu.sync_copy(wrows_v, outw_hbm.at[pl.ds(base, NB)])
        pltpu.sync_copy(cbuf_v, outc_hbm.at[pl.ds(base * H2, NB * H2)])
        return carry

    lax.fori_loop(0, NCHUNK, chunk_body, 0)


def kernel(w_idxs, c_idxs, word_table, char_table, word_proj,
           char_conv_w, char_conv_b):
    pw = _project_word(word_table, word_proj)
    pct = _char_tables(char_table, char_conv_w, char_conv_b)
    pct = jnp.pad(pct, ((0, 0), (0, PCT_STRIDE - H2)))
    out_w, out_c = _build_sc_main()(w_idxs.reshape(-1), c_idxs.reshape(-1),
                                    pw, pct.reshape(-1))
    return jnp.concatenate([out_w.reshape(B, L, H2),
                            out_c.reshape(B, L, H2)], axis=-1)


# speedup vs baseline: 9.7882x; 1.3315x over previous
import functools

import jax
import jax.numpy as jnp
from jax import lax
from jax.experimental import pallas as pl
from jax.experimental.pallas import tpu as pltpu
from jax.experimental.pallas import tpu_sc as plsc

VOCAB = 100000
WORD_DIM = 300
CHAR_VOCAB = 262
CHAR_DIM = 64
HIDDEN = 128
H2 = HIDDEN // 2
B = 1024
L = 200
W = 16
K = 5
T = W - K + 1

N = B * L

NC = 2
NS = 16
NW = NC * NS
LANES = 16

TOK_PER_W = N // NW
NB = 64
NCHUNK = TOK_PER_W // NB
NG = NB // LANES

ROWS_PCT = K * CHAR_VOCAB
HP = H2 // 2
PCT_STRIDE = HP + 1
CT_STRIDE = LANES + 1


def _pw_body(wt_ref, wp_ref, o_ref):
    o_ref[...] = jnp.dot(wt_ref[...], wp_ref[...],
                         preferred_element_type=jnp.float32)


def _project_word(word_table, word_proj):
    rows = 1000
    return pl.pallas_call(
        _pw_body,
        grid=(VOCAB // rows,),
        in_specs=[
            pl.BlockSpec((rows, WORD_DIM), lambda i: (i, 0)),
            pl.BlockSpec((WORD_DIM, H2), lambda i: (0, 0)),
        ],
        out_specs=pl.BlockSpec((rows, H2), lambda i: (i, 0)),
        out_shape=jax.ShapeDtypeStruct((VOCAB, H2), jnp.float32),
    )(word_table, word_proj)


def _pct_body(ct_ref, w_ref, b_ref, o_ref):
    k = pl.program_id(0)
    acc = jnp.dot(ct_ref[...], w_ref[0], preferred_element_type=jnp.float32)
    scale = jnp.where(k == 0, 1.0, 0.0)
    o_ref[0] = acc + scale * b_ref[...]


def _char_tables(char_table, char_conv_w, char_conv_b):
    out = pl.pallas_call(
        _pct_body,
        grid=(K,),
        in_specs=[
            pl.BlockSpec((CHAR_VOCAB, CHAR_DIM), lambda k: (0, 0)),
            pl.BlockSpec((1, CHAR_DIM, H2), lambda k: (k, 0, 0)),
            pl.BlockSpec((1, H2), lambda k: (0, 0)),
        ],
        out_specs=pl.BlockSpec((1, CHAR_VOCAB, H2), lambda k: (k, 0, 0)),
        out_shape=jax.ShapeDtypeStruct((K, CHAR_VOCAB, H2), jnp.float32),
    )(char_table, char_conv_w, char_conv_b.reshape(1, H2))
    return out.reshape(ROWS_PCT, H2)


@functools.cache
def _build_sc_main():
    mesh = plsc.VectorSubcoreMesh(core_axis_name="c", subcore_axis_name="s",
                                  num_cores=NC, num_subcores=NS)
    return pl.kernel(
        _sc_body,
        out_type=(jax.ShapeDtypeStruct((N, H2), jnp.float32),
                  jax.ShapeDtypeStruct((N * H2,), jnp.float32)),
        mesh=mesh,
        scratch_types=[
            pltpu.VMEM((ROWS_PCT * PCT_STRIDE,), jnp.float32),
            pltpu.VMEM((NB,), jnp.int32),
            pltpu.VMEM((NB * W,), jnp.int32),
            pltpu.VMEM((LANES * CT_STRIDE,), jnp.int32),
            pltpu.VMEM((NB, H2), jnp.float32),
            pltpu.VMEM((NB * H2,), jnp.float32),
            pltpu.SemaphoreType.DMA,
        ],
        compiler_params=pltpu.CompilerParams(use_tc_tiling_on_sc=False,
                                             needs_layout_passes=False),
    )


def _sc_body(wflat_hbm, cflat_hbm, pw_hbm, pct_hbm, outw_hbm, outc_hbm,
             pct_v, widx_v, cidx_v, ct_v, wrows_v, cbuf_v, sem):
    wid = lax.axis_index("s") * NC + lax.axis_index("c")
    pltpu.sync_copy(pct_hbm, pct_v)
    iota = lax.iota(jnp.int32, LANES)

    def chunk_body(ci, carry):
        base = wid * TOK_PER_W + ci * NB
        pltpu.sync_copy(wflat_hbm.at[pl.ds(base, NB)], widx_v)
        pltpu.sync_copy(cflat_hbm.at[pl.ds(base * W, NB * W)], cidx_v)
        pltpu.async_copy(pw_hbm.at[widx_v], wrows_v, sem).wait()

        def group_body(g, carry2):
            for i in range(LANES):
                chars = cidx_v[pl.ds((g * LANES + i) * W, W)]
                plsc.store_scatter(ct_v, [iota * CT_STRIDE + i], chars)
            cvec = [ct_v[pl.ds(j * CT_STRIDE, LANES)] * PCT_STRIDE
                    for j in range(W)]
            row = (g * LANES + iota) * H2

            def h_body(hp, carry3):
                m = None
                for t in range(T):
                    s = plsc.bitcast(
                        plsc.load_gather(pct_v, [cvec[t] + hp]), jnp.bfloat16)
                    for k in range(1, K):
                        s = s + plsc.bitcast(
                            plsc.load_gather(
                                pct_v,
                                [cvec[t + k]
                                 + (k * CHAR_VOCAB * PCT_STRIDE + hp)]),
                            jnp.bfloat16)
                    s = jnp.maximum(s, jnp.bfloat16(0))
                    m = s if m is None else jnp.maximum(m, s)
                a, b = plsc.unpack(m, format=plsc.PackFormat.INTERLEAVED)
                plsc.store_scatter(cbuf_v, [row + 2 * hp], a)
                plsc.store_scatter(cbuf_v, [row + 2 * hp + 1], b)
                return carry3

            lax.fori_loop(0, HP, h_body, 0)
            return carry2

        lax.fori_loop(0, NG, group_body, 0)

        pltpu.sync_copy(wrows_v, outw_hbm.at[pl.ds(base, NB)])
        pltpu.sync_copy(cbuf_v, outc_hbm.at[pl.ds(base * H2, NB * H2)])
        return carry

    lax.fori_loop(0, NCHUNK, chunk_body, 0)


def kernel(w_idxs, c_idxs, word_table, char_table, word_proj,
           char_conv_w, char_conv_b):
    pw = _project_word(word_table, word_proj)
    pct = _char_tables(char_table, char_conv_w, char_conv_b)
    pct = lax.bitcast_convert_type(
        pct.astype(jnp.bfloat16).reshape(ROWS_PCT, HP, 2), jnp.float32)
    pct = jnp.pad(pct, ((0, 0), (0, PCT_STRIDE - HP)))
    out_w, out_c = _build_sc_main()(w_idxs.reshape(-1), c_idxs.reshape(-1),
                                    pw, pct.reshape(-1))
    return jnp.concatenate([out_w.reshape(B, L, H2),
                            out_c.reshape(B, L, H2)], axis=-1)
